# baseline (speedup 1.0000x reference)
import jax
import jax.numpy as jnp
from jax import lax
from jax.experimental import pallas as pl
from jax.experimental.pallas import tpu as pltpu
from jax.experimental.pallas import tpu_sc as plsc

NUM_PATCHES = 1024
D = 768
B = 64
NUM_WORKERS = 32
F_PER_W = D // NUM_WORKERS
LANES = 16
VECS_PER_ROW = D // LANES
IDX_PER_W = B * F_PER_W
RB = 24
NBLK = IDX_PER_W // RB
NBUF = 4
SUB = 8


def _emb_body(x_hbm, tok_hbm, pos_hbm, out_hbm, pos_v, idx_all, *bufs):
    rows = bufs[:NBUF]
    gsem = bufs[NBUF:2 * NBUF]
    ssem = bufs[2 * NBUF:]

    c = lax.axis_index("c")
    s = lax.axis_index("s")
    wid = s * 2 + c
    f0 = wid * F_PER_W

    pltpu.sync_copy(pos_hbm.at[pl.ds(f0, F_PER_W)], pos_v)
    pltpu.sync_copy(x_hbm.at[pl.ds(wid * IDX_PER_W, IDX_PER_W)], idx_all)

    def idx_slice(m):
        return idx_all.at[pl.ds(m * RB, RB)]

    def out_slice(m):
        n0 = m * RB
        b = n0 // F_PER_W
        r0 = n0 % F_PER_W
        return out_hbm.at[pl.ds(b * D + f0 + r0, RB)], r0

    def launch_gather(m, k):
        for h in range(RB // SUB):
            pltpu.async_copy(
                tok_hbm.at[idx_all.at[pl.ds(m * RB + h * SUB, SUB)]],
                rows[k].at[pl.ds(h * SUB, SUB)], gsem[k])

    for k in range(NBUF - 1):
        launch_gather(k, k)

    def step(i, k):
        m = NBUF * i + k
        cur = rows[k]
        prv = rows[(k + NBUF - 1) % NBUF]

        @pl.when(m >= 1)
        def _():
            dst, _ = out_slice(m - 1)
            pltpu.make_async_copy(prv, dst, ssem[(k + NBUF - 1) % NBUF]).wait()

        @pl.when(m + NBUF - 1 < NBLK)
        def _():
            launch_gather(m + NBUF - 1, (k + NBUF - 1) % NBUF)

        n0 = m * RB
        b = n0 // F_PER_W
        r0 = n0 % F_PER_W
        for h in range(RB // SUB):
            pltpu.make_async_copy(
                tok_hbm.at[idx_all.at[pl.ds(h * SUB, SUB)]],
                cur.at[pl.ds(h * SUB, SUB)], gsem[k]).wait()

            @pl.loop(h * SUB, (h + 1) * SUB)
            def _(r):
                for j in range(VECS_PER_ROW):
                    sl = pl.ds(j * LANES, LANES)
                    plsc.addupdate(cur.at[r, sl], pos_v[r0 + r, sl])

            pltpu.async_copy(
                cur.at[pl.ds(h * SUB, SUB)],
                out_hbm.at[pl.ds(b * D + f0 + r0 + h * SUB, SUB)], ssem[k])

    def body(i, carry):
        for k in range(NBUF):
            step(i, k)
        return carry

    lax.fori_loop(0, NBLK // NBUF, body, 0)
    dst_last, _ = out_slice(NBLK - 1)
    pltpu.make_async_copy(rows[(NBLK - 1) % NBUF], dst_last,
                          ssem[(NBLK - 1) % NBUF]).wait()


@jax.jit
def kernel(x, token_table, pos_table):
    xp = x.reshape(B, NUM_WORKERS, F_PER_W).transpose(1, 0, 2).reshape(-1)
    mesh = plsc.VectorSubcoreMesh(core_axis_name="c", subcore_axis_name="s")
    scratch = (
        [pltpu.VMEM((F_PER_W, D), jnp.float32),
         pltpu.VMEM((IDX_PER_W,), jnp.int32)]
        + [pltpu.VMEM((RB, D), jnp.float32) for _ in range(NBUF)]
        + [pltpu.SemaphoreType.DMA for _ in range(2 * NBUF)]
    )
    out = pl.kernel(
        _emb_body,
        out_type=jax.ShapeDtypeStruct((B * D, D), jnp.float32),
        mesh=mesh,
        scratch_types=scratch,
    )(xp, token_table, pos_table)
    return out.reshape(B, D, D)


# speedup vs baseline: 1.4162x; 1.4162x over previous
import jax
import jax.numpy as jnp
from jax import lax
from jax.experimental import pallas as pl
from jax.experimental.pallas import tpu as pltpu
from jax.experimental.pallas import tpu_sc as plsc

NUM_PATCHES = 1024
D = 768
B = 64
NUM_WORKERS = 32
F_PER_W = D // NUM_WORKERS
LANES = 16
VECS_PER_ROW = D // LANES
IDX_PER_W = B * F_PER_W
RB = 24
NBLK = IDX_PER_W // RB
NBUF = 2
SUB = 8


def _emb_body(x_hbm, tok_hbm, pos_hbm, out_hbm, pos_v, idx_all, tok_sp,
              *bufs):
    rows = bufs[:NBUF]
    gsem = bufs[NBUF:2 * NBUF]
    rsem = bufs[2 * NBUF:3 * NBUF]
    ssem = bufs[3 * NBUF:]

    c = lax.axis_index("c")
    s = lax.axis_index("s")
    wid = s * 2 + c
    f0 = wid * F_PER_W

    rpt = NUM_PATCHES // 16
    pltpu.sync_copy(tok_hbm.at[pl.ds(s * rpt, rpt)],
                    tok_sp.at[pl.ds(s * rpt, rpt)])

    pltpu.sync_copy(pos_hbm.at[pl.ds(f0, F_PER_W)], pos_v)
    pltpu.sync_copy(x_hbm.at[pl.ds(wid * IDX_PER_W, IDX_PER_W)], idx_all)

    plsc.subcore_barrier()

    def idx_slice(m):
        return idx_all.at[pl.ds(m * RB, RB)]

    def out_slice(m):
        n0 = m * RB
        b = n0 // F_PER_W
        r0 = n0 % F_PER_W
        return out_hbm.at[pl.ds(b * D + f0 + r0, RB)], r0

    def launch_gather(m, k):
        for h in range(2):
            pltpu.async_copy(
                tok_hbm.at[idx_all.at[pl.ds(m * RB + h * SUB, SUB)]],
                rows[k].at[pl.ds(h * SUB, SUB)], gsem[k])
        v1 = idx_all[pl.ds(m * RB + 8, LANES)]
        for r in range(2 * SUB, RB):
            pltpu.async_copy(tok_sp.at[pl.ds(v1[r - 8], 1)],
                             rows[k].at[pl.ds(r, 1)], rsem[k])

    for k in range(NBUF - 1):
        launch_gather(k, k)

    def step(i, k):
        m = NBUF * i + k
        cur = rows[k]
        prv = rows[(k + NBUF - 1) % NBUF]

        @pl.when(m >= 1)
        def _():
            dst, _ = out_slice(m - 1)
            pltpu.make_async_copy(prv, dst, ssem[(k + NBUF - 1) % NBUF]).wait()

        @pl.when(m + NBUF - 1 < NBLK)
        def _():
            launch_gather(m + NBUF - 1, (k + NBUF - 1) % NBUF)

        n0 = m * RB
        b = n0 // F_PER_W
        r0 = n0 % F_PER_W
        for h in range(RB // SUB):
            if h < 2:
                pltpu.make_async_copy(
                    tok_hbm.at[idx_all.at[pl.ds(h * SUB, SUB)]],
                    cur.at[pl.ds(h * SUB, SUB)], gsem[k]).wait()
            else:
                pltpu.make_async_copy(
                    tok_sp.at[pl.ds(0, SUB)],
                    cur.at[pl.ds(h * SUB, SUB)], rsem[k]).wait()

            @pl.loop(h * SUB, (h + 1) * SUB)
            def _(r):
                for j in range(VECS_PER_ROW):
                    sl = pl.ds(j * LANES, LANES)
                    plsc.addupdate(cur.at[r, sl], pos_v[r0 + r, sl])

            pltpu.async_copy(
                cur.at[pl.ds(h * SUB, SUB)],
                out_hbm.at[pl.ds(b * D + f0 + r0 + h * SUB, SUB)], ssem[k])

    def body(i, carry):
        for k in range(NBUF):
            step(i, k)
        return carry

    lax.fori_loop(0, NBLK // NBUF, body, 0)
    dst_last, _ = out_slice(NBLK - 1)
    pltpu.make_async_copy(rows[(NBLK - 1) % NBUF], dst_last,
                          ssem[(NBLK - 1) % NBUF]).wait()


@jax.jit
def kernel(x, token_table, pos_table):
    xp = x.reshape(B, NUM_WORKERS, F_PER_W).transpose(1, 0, 2).reshape(-1)
    mesh = plsc.VectorSubcoreMesh(core_axis_name="c", subcore_axis_name="s")
    scratch = (
        [pltpu.VMEM((F_PER_W, D), jnp.float32),
         pltpu.VMEM((IDX_PER_W,), jnp.int32),
         pltpu.VMEM_SHARED((NUM_PATCHES, D), jnp.float32)]
        + [pltpu.VMEM((RB, D), jnp.float32) for _ in range(NBUF)]
        + [pltpu.SemaphoreType.DMA for _ in range(3 * NBUF)]
    )
    out = pl.kernel(
        _emb_body,
        out_type=jax.ShapeDtypeStruct((B * D, D), jnp.float32),
        mesh=mesh,
        scratch_types=scratch,
    )(xp, token_table, pos_table)
    return out.reshape(B, D, D)
